# baseline (speedup 1.0000x reference)
import functools

import jax
import jax.numpy as jnp
from jax import lax
from jax.experimental import pallas as pl
from jax.experimental.pallas import tpu as pltpu
from jax.experimental.pallas import tpu_sc as plsc

NUM_CODES = 8192
EMBED_DIM = 256
COMMITMENT_COST = 0.25
M_BLK = 128


def _argmin_body(z_ref, cb_ref, idx_ref, perp_ref, cnt_ref):
    i = pl.program_id(0)
    nsteps = pl.num_programs(0)
    zb = z_ref[...]
    cb = cb_ref[...]
    mm = jax.lax.dot_general(
        zb, cb, (((1,), (1,)), ((), ())), preferred_element_type=jnp.float32
    )
    z2 = jnp.sum(zb * zb, axis=1, keepdims=True)
    c2 = jnp.sum(cb * cb, axis=1)
    dist = z2 - 2.0 * mm + c2[None, :]
    mv = jnp.min(dist, axis=1, keepdims=True)
    kiota = jax.lax.broadcasted_iota(jnp.int32, (M_BLK, NUM_CODES), 1)
    idx = jnp.min(
        jnp.where(dist == mv, kiota, jnp.int32(NUM_CODES)), axis=1
    ).astype(jnp.int32)
    idx_ref[0, 0, :] = idx
    oh = (kiota == idx[:, None]).astype(jnp.float32)
    pc = jnp.sum(oh, axis=0)

    @pl.when(i == 0)
    def _init():
        cnt_ref[...] = pc[None, :]

    @pl.when(i > 0)
    def _acc():
        cnt_ref[...] += pc[None, :]

    @pl.when(i == nsteps - 1)
    def _finish():
        n_tok = nsteps * M_BLK
        p = cnt_ref[...] * (1.0 / n_tok)
        ent = -jnp.sum(p * jnp.log(p + 1e-10))
        perp_ref[...] = jnp.exp(ent)[None, None]


def _st_body(z_ref, q_ref, out_ref, loss_ref):
    i = pl.program_id(0)
    nsteps = pl.num_programs(0)
    zb = z_ref[...]
    qb = q_ref[...]
    d = qb - zb
    out_ref[...] = zb + d
    psum = jnp.sum(d * d)

    @pl.when(i == 0)
    def _init():
        loss_ref[...] = psum[None, None]

    @pl.when(i > 0)
    def _acc():
        loss_ref[...] += psum[None, None]

    @pl.when(i == nsteps - 1)
    def _finish():
        n = nsteps * M_BLK * EMBED_DIM
        loss_ref[...] = (COMMITMENT_COST / n) * loss_ref[...]


def _make_sc_gather(n_tok):
    info = plsc.get_sparse_core_info()
    nw = info.num_cores * info.num_subcores
    b_per_w = n_tok // nw
    chunk = 128
    n_chunks = b_per_w // chunk
    mesh = plsc.VectorSubcoreMesh(core_axis_name="c", subcore_axis_name="s")

    @functools.partial(
        pl.kernel, mesh=mesh,
        out_type=jax.ShapeDtypeStruct((n_tok, EMBED_DIM), jnp.float32),
        scratch_types=[
            pltpu.VMEM((chunk,), jnp.int32),
            pltpu.VMEM((chunk, EMBED_DIM), jnp.float32),
            pltpu.SemaphoreType.DMA,
        ],
    )
    def gather_k(table_hbm, idx_hbm, out_hbm, idx_v, rows_v, sem):
        wid = lax.axis_index("s") * info.num_cores + lax.axis_index("c")
        for j in range(n_chunks):
            base = wid * b_per_w + j * chunk
            pltpu.sync_copy(idx_hbm.at[pl.ds(base, chunk)], idx_v)
            pltpu.async_copy(table_hbm.at[idx_v], rows_v, sem).wait()
            pltpu.sync_copy(rows_v, out_hbm.at[pl.ds(base, chunk)])

    return gather_k


def kernel(z, codebook):
    D = z.shape[-1]
    z_flat = z.reshape(-1, D)
    n_tok = z_flat.shape[0]
    grid = n_tok // M_BLK

    idx3, perp, _counts = pl.pallas_call(
        _argmin_body,
        grid=(grid,),
        in_specs=[
            pl.BlockSpec((M_BLK, D), lambda i: (i, 0)),
            pl.BlockSpec((NUM_CODES, D), lambda i: (0, 0)),
        ],
        out_specs=[
            pl.BlockSpec((1, 1, M_BLK), lambda i: (i, 0, 0)),
            pl.BlockSpec((1, 1), lambda i: (0, 0)),
            pl.BlockSpec((1, NUM_CODES), lambda i: (0, 0)),
        ],
        out_shape=[
            jax.ShapeDtypeStruct((grid, 1, M_BLK), jnp.int32),
            jax.ShapeDtypeStruct((1, 1), jnp.float32),
            jax.ShapeDtypeStruct((1, NUM_CODES), jnp.float32),
        ],
    )(z_flat, codebook)
    idx_flat = idx3.reshape(-1)

    q_flat = _make_sc_gather(n_tok)(codebook, idx_flat)

    qst, loss = pl.pallas_call(
        _st_body,
        grid=(grid,),
        in_specs=[
            pl.BlockSpec((M_BLK, D), lambda i: (i, 0)),
            pl.BlockSpec((M_BLK, D), lambda i: (i, 0)),
        ],
        out_specs=[
            pl.BlockSpec((M_BLK, D), lambda i: (i, 0)),
            pl.BlockSpec((1, 1), lambda i: (0, 0)),
        ],
        out_shape=[
            jax.ShapeDtypeStruct((n_tok, D), jnp.float32),
            jax.ShapeDtypeStruct((1, 1), jnp.float32),
        ],
    )(z_flat, q_flat)

    return (
        qst.reshape(z.shape),
        loss[0, 0],
        idx_flat.reshape(z.shape[:-1]),
        perp[0, 0],
    )


# speedup vs baseline: 1.0834x; 1.0834x over previous
import functools

import jax
import jax.numpy as jnp
from jax import lax
from jax.experimental import pallas as pl
from jax.experimental.pallas import tpu as pltpu
from jax.experimental.pallas import tpu_sc as plsc

NUM_CODES = 8192
EMBED_DIM = 256
COMMITMENT_COST = 0.25
M_BLK = 128


def _norms_body(cb_ref, c2_ref):
    cb = cb_ref[...]
    c2_ref[...] = jnp.sum(cb * cb, axis=1)[None, :]


def _argmin_body(z_ref, cb_ref, c2_ref, idx_ref, perp_ref, cnt_ref):
    i = pl.program_id(0)
    nsteps = pl.num_programs(0)
    zb = z_ref[...]
    cb = cb_ref[...]
    mm = jax.lax.dot_general(
        zb, cb, (((1,), (1,)), ((), ())), preferred_element_type=jnp.float32
    )
    dist = c2_ref[...] - 2.0 * mm
    mv = jnp.min(dist, axis=1, keepdims=True)
    kiota = jax.lax.broadcasted_iota(jnp.int32, (M_BLK, NUM_CODES), 1)
    idx = jnp.min(
        jnp.where(dist == mv, kiota, jnp.int32(NUM_CODES)), axis=1
    ).astype(jnp.int32)
    idx_ref[0, 0, :] = idx
    oh = (kiota == idx[:, None]).astype(jnp.float32)
    pc = jnp.sum(oh, axis=0)

    @pl.when(i == 0)
    def _init():
        cnt_ref[...] = pc[None, :]

    @pl.when(i > 0)
    def _acc():
        cnt_ref[...] += pc[None, :]

    @pl.when(i == nsteps - 1)
    def _finish():
        n_tok = nsteps * M_BLK
        p = cnt_ref[...] * (1.0 / n_tok)
        ent = -jnp.sum(p * jnp.log(p + 1e-10))
        perp_ref[...] = jnp.exp(ent)[None, None]


def _st_body(z_ref, q_ref, out_ref, loss_ref):
    i = pl.program_id(0)
    nsteps = pl.num_programs(0)
    zb = z_ref[...]
    qb = q_ref[...]
    d = qb - zb
    out_ref[...] = zb + d
    psum = jnp.sum(d * d)

    @pl.when(i == 0)
    def _init():
        loss_ref[...] = psum[None, None]

    @pl.when(i > 0)
    def _acc():
        loss_ref[...] += psum[None, None]

    @pl.when(i == nsteps - 1)
    def _finish():
        n = nsteps * M_BLK * EMBED_DIM
        loss_ref[...] = (COMMITMENT_COST / n) * loss_ref[...]


def _make_sc_gather(n_tok):
    info = plsc.get_sparse_core_info()
    nw = info.num_cores * info.num_subcores
    b_per_w = n_tok // nw
    chunk = 128
    n_chunks = b_per_w // chunk
    mesh = plsc.VectorSubcoreMesh(core_axis_name="c", subcore_axis_name="s")

    @functools.partial(
        pl.kernel, mesh=mesh,
        out_type=jax.ShapeDtypeStruct((n_tok, EMBED_DIM), jnp.float32),
        scratch_types=[
            pltpu.VMEM((chunk,), jnp.int32),
            pltpu.VMEM((chunk, EMBED_DIM), jnp.float32),
            pltpu.SemaphoreType.DMA,
        ],
    )
    def gather_k(table_hbm, idx_hbm, out_hbm, idx_v, rows_v, sem):
        wid = lax.axis_index("s") * info.num_cores + lax.axis_index("c")
        for j in range(n_chunks):
            base = wid * b_per_w + j * chunk
            pltpu.sync_copy(idx_hbm.at[pl.ds(base, chunk)], idx_v)
            pltpu.async_copy(table_hbm.at[idx_v], rows_v, sem).wait()
            pltpu.sync_copy(rows_v, out_hbm.at[pl.ds(base, chunk)])

    return gather_k


def kernel(z, codebook):
    D = z.shape[-1]
    z_flat = z.reshape(-1, D)
    n_tok = z_flat.shape[0]
    grid = n_tok // M_BLK

    c2 = pl.pallas_call(
        _norms_body,
        in_specs=[pl.BlockSpec((NUM_CODES, D), lambda: (0, 0))],
        out_specs=pl.BlockSpec((1, NUM_CODES), lambda: (0, 0)),
        out_shape=jax.ShapeDtypeStruct((1, NUM_CODES), jnp.float32),
    )(codebook)

    idx3, perp, _counts = pl.pallas_call(
        _argmin_body,
        grid=(grid,),
        in_specs=[
            pl.BlockSpec((M_BLK, D), lambda i: (i, 0)),
            pl.BlockSpec((NUM_CODES, D), lambda i: (0, 0)),
            pl.BlockSpec((1, NUM_CODES), lambda i: (0, 0)),
        ],
        out_specs=[
            pl.BlockSpec((1, 1, M_BLK), lambda i: (i, 0, 0)),
            pl.BlockSpec((1, 1), lambda i: (0, 0)),
            pl.BlockSpec((1, NUM_CODES), lambda i: (0, 0)),
        ],
        out_shape=[
            jax.ShapeDtypeStruct((grid, 1, M_BLK), jnp.int32),
            jax.ShapeDtypeStruct((1, 1), jnp.float32),
            jax.ShapeDtypeStruct((1, NUM_CODES), jnp.float32),
        ],
    )(z_flat, codebook, c2)
    idx_flat = idx3.reshape(-1)

    q_flat = _make_sc_gather(n_tok)(codebook, idx_flat)

    qst, loss = pl.pallas_call(
        _st_body,
        grid=(grid,),
        in_specs=[
            pl.BlockSpec((M_BLK, D), lambda i: (i, 0)),
            pl.BlockSpec((M_BLK, D), lambda i: (i, 0)),
        ],
        out_specs=[
            pl.BlockSpec((M_BLK, D), lambda i: (i, 0)),
            pl.BlockSpec((1, 1), lambda i: (0, 0)),
        ],
        out_shape=[
            jax.ShapeDtypeStruct((n_tok, D), jnp.float32),
            jax.ShapeDtypeStruct((1, 1), jnp.float32),
        ],
    )(z_flat, q_flat)

    return (
        qst.reshape(z.shape),
        loss[0, 0],
        idx_flat.reshape(z.shape[:-1]),
        perp[0, 0],
    )


# speedup vs baseline: 1.2679x; 1.1703x over previous
import functools

import jax
import jax.numpy as jnp
from jax import lax
from jax.experimental import pallas as pl
from jax.experimental.pallas import tpu as pltpu
from jax.experimental.pallas import tpu_sc as plsc

NUM_CODES = 8192
EMBED_DIM = 256
COMMITMENT_COST = 0.25
M_BLK = 128


def _norms_body(cb_ref, c2_ref):
    cb = cb_ref[...]
    c2_ref[...] = jnp.sum(cb * cb, axis=1)[None, :]


def _argmin_body(z_ref, cb_ref, c2_ref, idx_ref, perp_ref, cnt_ref):
    i = pl.program_id(0)
    nsteps = pl.num_programs(0)
    zb = z_ref[...]
    cb = cb_ref[...]
    mm = jax.lax.dot_general(
        zb, cb, (((1,), (1,)), ((), ())), preferred_element_type=jnp.float32
    )
    dist = c2_ref[...] - 2.0 * mm
    idx = jnp.argmin(dist, axis=1).astype(jnp.int32)
    kiota = jax.lax.broadcasted_iota(jnp.int32, (M_BLK, NUM_CODES), 1)
    idx_ref[0, 0, :] = idx
    oh = (kiota == idx[:, None]).astype(jnp.float32)
    pc = jnp.sum(oh, axis=0)

    @pl.when(i == 0)
    def _init():
        cnt_ref[...] = pc[None, :]

    @pl.when(i > 0)
    def _acc():
        cnt_ref[...] += pc[None, :]

    @pl.when(i == nsteps - 1)
    def _finish():
        n_tok = nsteps * M_BLK
        p = cnt_ref[...] * (1.0 / n_tok)
        ent = -jnp.sum(p * jnp.log(p + 1e-10))
        perp_ref[...] = jnp.exp(ent)[None, None]


def _st_body(z_ref, q_ref, out_ref, loss_ref):
    i = pl.program_id(0)
    nsteps = pl.num_programs(0)
    zb = z_ref[...]
    qb = q_ref[...]
    d = qb - zb
    out_ref[...] = zb + d
    psum = jnp.sum(d * d)

    @pl.when(i == 0)
    def _init():
        loss_ref[...] = psum[None, None]

    @pl.when(i > 0)
    def _acc():
        loss_ref[...] += psum[None, None]

    @pl.when(i == nsteps - 1)
    def _finish():
        n = nsteps * M_BLK * EMBED_DIM
        loss_ref[...] = (COMMITMENT_COST / n) * loss_ref[...]


def _make_sc_gather(n_tok):
    info = plsc.get_sparse_core_info()
    nw = info.num_cores * info.num_subcores
    b_per_w = n_tok // nw
    chunk = 128
    n_chunks = b_per_w // chunk
    mesh = plsc.VectorSubcoreMesh(core_axis_name="c", subcore_axis_name="s")

    @functools.partial(
        pl.kernel, mesh=mesh,
        out_type=jax.ShapeDtypeStruct((n_tok, EMBED_DIM), jnp.float32),
        scratch_types=[
            pltpu.VMEM((chunk,), jnp.int32),
            pltpu.VMEM((chunk, EMBED_DIM), jnp.float32),
            pltpu.SemaphoreType.DMA,
        ],
    )
    def gather_k(table_hbm, idx_hbm, out_hbm, idx_v, rows_v, sem):
        wid = lax.axis_index("s") * info.num_cores + lax.axis_index("c")
        for j in range(n_chunks):
            base = wid * b_per_w + j * chunk
            pltpu.sync_copy(idx_hbm.at[pl.ds(base, chunk)], idx_v)
            pltpu.async_copy(table_hbm.at[idx_v], rows_v, sem).wait()
            pltpu.sync_copy(rows_v, out_hbm.at[pl.ds(base, chunk)])

    return gather_k


def kernel(z, codebook):
    D = z.shape[-1]
    z_flat = z.reshape(-1, D)
    n_tok = z_flat.shape[0]
    grid = n_tok // M_BLK

    c2 = pl.pallas_call(
        _norms_body,
        in_specs=[pl.BlockSpec((NUM_CODES, D), lambda: (0, 0))],
        out_specs=pl.BlockSpec((1, NUM_CODES), lambda: (0, 0)),
        out_shape=jax.ShapeDtypeStruct((1, NUM_CODES), jnp.float32),
    )(codebook)

    idx3, perp, _counts = pl.pallas_call(
        _argmin_body,
        grid=(grid,),
        in_specs=[
            pl.BlockSpec((M_BLK, D), lambda i: (i, 0)),
            pl.BlockSpec((NUM_CODES, D), lambda i: (0, 0)),
            pl.BlockSpec((1, NUM_CODES), lambda i: (0, 0)),
        ],
        out_specs=[
            pl.BlockSpec((1, 1, M_BLK), lambda i: (i, 0, 0)),
            pl.BlockSpec((1, 1), lambda i: (0, 0)),
            pl.BlockSpec((1, NUM_CODES), lambda i: (0, 0)),
        ],
        out_shape=[
            jax.ShapeDtypeStruct((grid, 1, M_BLK), jnp.int32),
            jax.ShapeDtypeStruct((1, 1), jnp.float32),
            jax.ShapeDtypeStruct((1, NUM_CODES), jnp.float32),
        ],
    )(z_flat, codebook, c2)
    idx_flat = idx3.reshape(-1)

    q_flat = _make_sc_gather(n_tok)(codebook, idx_flat)

    qst, loss = pl.pallas_call(
        _st_body,
        grid=(grid,),
        in_specs=[
            pl.BlockSpec((M_BLK, D), lambda i: (i, 0)),
            pl.BlockSpec((M_BLK, D), lambda i: (i, 0)),
        ],
        out_specs=[
            pl.BlockSpec((M_BLK, D), lambda i: (i, 0)),
            pl.BlockSpec((1, 1), lambda i: (0, 0)),
        ],
        out_shape=[
            jax.ShapeDtypeStruct((n_tok, D), jnp.float32),
            jax.ShapeDtypeStruct((1, 1), jnp.float32),
        ],
    )(z_flat, q_flat)

    return (
        qst.reshape(z.shape),
        loss[0, 0],
        idx_flat.reshape(z.shape[:-1]),
        perp[0, 0],
    )
